# baseline (speedup 1.0000x reference)
import jax
import jax.numpy as jnp
from jax import lax
from jax.experimental import pallas as pl
from jax.experimental.pallas import tpu as pltpu
from jax.experimental.pallas import tpu_sc as plsc

D = 128
H = 50
B = 4096
N_ROWS = 100000


_R_BLK = 16384


def _scores_body(wu_ref, wi_ref, bias_ref, u_ref, i_ref,
                 o0_ref, o1_ref, o2_ref, o3_ref, o4_ref):
    u = u_ref[...]
    it = i_ref[...]
    dn = (((1,), (1,)), ((), ()))
    res = (
        lax.dot_general(wu_ref[...], u, dn, preferred_element_type=jnp.float32)
        + lax.dot_general(wi_ref[...], it, dn, preferred_element_type=jnp.float32)
    )
    b = bias_ref[0, 0]
    o0_ref[...] = res[0, :] + b
    o1_ref[...] = res[1, :]
    o2_ref[...] = res[2, :]
    o3_ref[...] = res[3, :]
    o4_ref[...] = res[4, :]


def _scores_tc(user_table, item_table, wu, wi, bias_row):
    n_blk = (N_ROWS + _R_BLK - 1) // _R_BLK
    one_d = pl.BlockSpec((_R_BLK,), lambda i: (i,))
    return pl.pallas_call(
        _scores_body,
        grid=(n_blk,),
        in_specs=[
            pl.BlockSpec((8, D), lambda i: (0, 0)),
            pl.BlockSpec((8, D), lambda i: (0, 0)),
            pl.BlockSpec((1, 8), lambda i: (0, 0)),
            pl.BlockSpec((_R_BLK, D), lambda i: (i, 0)),
            pl.BlockSpec((_R_BLK, D), lambda i: (i, 0)),
        ],
        out_specs=[one_d] * 5,
        out_shape=[jax.ShapeDtypeStruct((N_ROWS,), jnp.float32)] * 5,
    )(wu, wi, bias_row, user_table, item_table)


_NC = 2
_NS = 16
_NW = _NC * _NS
_BPW = B // _NW
_NG = _BPW // 16


def _sc_body(uid_hbm, iid_hbm, pvf_hbm, byf_hbm, fvf_hbm,
             us_hbm, is_hbm, pvs_hbm, bys_hbm, fvs_hbm,
             out_hbm,
             uidx, iidx, pvidx, byidx, fvidx,
             uval, ival, pvval, byval, fvval, obuf, sem):
    wid = lax.axis_index("s") * _NC + lax.axis_index("c")
    base = wid * _BPW
    hbase = base * H
    pltpu.sync_copy(uid_hbm.at[pl.ds(base, _BPW)], uidx)
    pltpu.sync_copy(iid_hbm.at[pl.ds(base, _BPW)], iidx)
    pltpu.sync_copy(pvf_hbm.at[pl.ds(hbase, _BPW * H)], pvidx)
    pltpu.sync_copy(byf_hbm.at[pl.ds(hbase, _BPW * H)], byidx)
    pltpu.sync_copy(fvf_hbm.at[pl.ds(hbase, _BPW * H)], fvidx)
    c0 = pltpu.async_copy(us_hbm.at[uidx], uval, sem)
    c1 = pltpu.async_copy(is_hbm.at[iidx], ival, sem)
    c2 = pltpu.async_copy(pvs_hbm.at[pvidx], pvval, sem)
    c3 = pltpu.async_copy(bys_hbm.at[byidx], byval, sem)
    c4 = pltpu.async_copy(fvs_hbm.at[fvidx], fvval, sem)
    c0.wait(); c1.wait(); c2.wait(); c3.wait(); c4.wait()
    inv_h = jnp.float32(1.0 / H)
    lanes = lax.iota(jnp.int32, 16)
    for g in range(_NG):
        sl = pl.ds(g * 16, 16)
        bvec = (g * 16 + lanes) * H

        def hbody(h, acc):
            idx = bvec + h
            return (acc
                    + plsc.load_gather(pvval, [idx])
                    + plsc.load_gather(byval, [idx])
                    + plsc.load_gather(fvval, [idx]))

        acc = lax.fori_loop(0, H, hbody, jnp.zeros((16,), jnp.float32))
        x = uval[sl] + ival[sl] + acc * inv_h
        obuf[sl] = 1.0 / (1.0 + jnp.exp(-x))
    pltpu.sync_copy(obuf, out_hbm.at[pl.ds(base, _BPW)])


def _sc_pool(user_id, item_id, pvf, byf, fvf, us, is_, pvs, bys, fvs):
    mesh = plsc.VectorSubcoreMesh(core_axis_name="c", subcore_axis_name="s",
                                  num_cores=_NC, num_subcores=_NS)
    run = pl.kernel(
        _sc_body,
        jax.ShapeDtypeStruct((B,), jnp.float32),
        mesh=mesh,
        compiler_params=pltpu.CompilerParams(needs_layout_passes=False),
        scratch_types=[
            pltpu.VMEM((_BPW,), jnp.int32),
            pltpu.VMEM((_BPW,), jnp.int32),
            pltpu.VMEM((H * _BPW,), jnp.int32),
            pltpu.VMEM((H * _BPW,), jnp.int32),
            pltpu.VMEM((H * _BPW,), jnp.int32),
            pltpu.VMEM((_BPW,), jnp.float32),
            pltpu.VMEM((_BPW,), jnp.float32),
            pltpu.VMEM((H * _BPW,), jnp.float32),
            pltpu.VMEM((H * _BPW,), jnp.float32),
            pltpu.VMEM((H * _BPW,), jnp.float32),
            pltpu.VMEM((_BPW,), jnp.float32),
            pltpu.SemaphoreType.DMA,
        ],
    )
    return run(user_id, item_id, pvf, byf, fvf, us, is_, pvs, bys, fvs)


def kernel(user_id, item_id, pv_history, buy_history, fav_history,
           user_table, item_table, fc_w, fc_b):
    w = fc_w[0]
    zrow = jnp.zeros((1, D), jnp.float32)
    wu = jnp.concatenate([w[0:128][None, :]] + [zrow] * 7, axis=0)
    wi = jnp.concatenate(
        [zrow, w[128:256][None, :], w[256:384][None, :],
         w[384:512][None, :], w[512:640][None, :], zrow, zrow, zrow], axis=0)
    bias_row = jnp.zeros((1, 8), jnp.float32).at[0, 0].set(fc_b[0])
    us, is_, pvs, bys, fvs = _scores_tc(user_table, item_table, wu, wi, bias_row)
    return _sc_pool(user_id, item_id,
                    pv_history.reshape(-1), buy_history.reshape(-1),
                    fav_history.reshape(-1), us, is_, pvs, bys, fvs)


# speedup vs baseline: 1.0406x; 1.0406x over previous
import jax
import jax.numpy as jnp
from jax import lax
from jax.experimental import pallas as pl
from jax.experimental.pallas import tpu as pltpu
from jax.experimental.pallas import tpu_sc as plsc

D = 128
H = 50
B = 4096
N_ROWS = 100000


_R_BLK = 10240


def _scores_body(wu_ref, wi_ref, bias_ref, u_ref, i_ref,
                 o0_ref, o1_ref, o2_ref, o3_ref, o4_ref):
    u = u_ref[...]
    it = i_ref[...]
    dn = (((1,), (1,)), ((), ()))
    res = (
        lax.dot_general(wu_ref[...], u, dn, preferred_element_type=jnp.float32)
        + lax.dot_general(wi_ref[...], it, dn, preferred_element_type=jnp.float32)
    )
    b = bias_ref[0, 0]
    o0_ref[...] = res[0, :] + b
    o1_ref[...] = res[1, :]
    o2_ref[...] = res[2, :]
    o3_ref[...] = res[3, :]
    o4_ref[...] = res[4, :]


def _scores_tc(user_table, item_table, wu, wi, bias_row):
    n_blk = (N_ROWS + _R_BLK - 1) // _R_BLK
    one_d = pl.BlockSpec((_R_BLK,), lambda i: (i,))
    return pl.pallas_call(
        _scores_body,
        grid=(n_blk,),
        in_specs=[
            pl.BlockSpec((8, D), lambda i: (0, 0)),
            pl.BlockSpec((8, D), lambda i: (0, 0)),
            pl.BlockSpec((1, 8), lambda i: (0, 0)),
            pl.BlockSpec((_R_BLK, D), lambda i: (i, 0)),
            pl.BlockSpec((_R_BLK, D), lambda i: (i, 0)),
        ],
        out_specs=[one_d] * 5,
        out_shape=[jax.ShapeDtypeStruct((N_ROWS,), jnp.float32)] * 5,
    )(wu, wi, bias_row, user_table, item_table)


_NC = 2
_NS = 16
_NW = _NC * _NS
_BPW = B // _NW
_NG = _BPW // 16


def _sc_body(uid_hbm, iid_hbm, pvf_hbm, byf_hbm, fvf_hbm,
             us_hbm, is_hbm, pvs_hbm, bys_hbm, fvs_hbm,
             out_hbm,
             uidx, iidx, pvidx, byidx, fvidx,
             uval, ival, pvval, byval, fvval, obuf, sem):
    wid = lax.axis_index("s") * _NC + lax.axis_index("c")
    base = wid * _BPW
    hbase = base * H
    pltpu.sync_copy(uid_hbm.at[pl.ds(base, _BPW)], uidx)
    pltpu.sync_copy(iid_hbm.at[pl.ds(base, _BPW)], iidx)
    pltpu.sync_copy(pvf_hbm.at[pl.ds(hbase, _BPW * H)], pvidx)
    pltpu.sync_copy(byf_hbm.at[pl.ds(hbase, _BPW * H)], byidx)
    pltpu.sync_copy(fvf_hbm.at[pl.ds(hbase, _BPW * H)], fvidx)
    c0 = pltpu.async_copy(us_hbm.at[uidx], uval, sem)
    c1 = pltpu.async_copy(is_hbm.at[iidx], ival, sem)
    c2 = pltpu.async_copy(pvs_hbm.at[pvidx], pvval, sem)
    c3 = pltpu.async_copy(bys_hbm.at[byidx], byval, sem)
    c4 = pltpu.async_copy(fvs_hbm.at[fvidx], fvval, sem)
    c0.wait(); c1.wait(); c2.wait(); c3.wait(); c4.wait()
    inv_h = jnp.float32(1.0 / H)
    lanes = lax.iota(jnp.int32, 16)
    for g in range(_NG):
        sl = pl.ds(g * 16, 16)
        bvec = (g * 16 + lanes) * H

        def hbody(h, acc):
            idx = bvec + h
            return (acc
                    + plsc.load_gather(pvval, [idx])
                    + plsc.load_gather(byval, [idx])
                    + plsc.load_gather(fvval, [idx]))

        acc = lax.fori_loop(0, H, hbody, jnp.zeros((16,), jnp.float32))
        x = uval[sl] + ival[sl] + acc * inv_h
        obuf[sl] = 1.0 / (1.0 + jnp.exp(-x))
    pltpu.sync_copy(obuf, out_hbm.at[pl.ds(base, _BPW)])


def _sc_pool(user_id, item_id, pvf, byf, fvf, us, is_, pvs, bys, fvs):
    mesh = plsc.VectorSubcoreMesh(core_axis_name="c", subcore_axis_name="s",
                                  num_cores=_NC, num_subcores=_NS)
    run = pl.kernel(
        _sc_body,
        jax.ShapeDtypeStruct((B,), jnp.float32),
        mesh=mesh,
        compiler_params=pltpu.CompilerParams(needs_layout_passes=False),
        scratch_types=[
            pltpu.VMEM((_BPW,), jnp.int32),
            pltpu.VMEM((_BPW,), jnp.int32),
            pltpu.VMEM((H * _BPW,), jnp.int32),
            pltpu.VMEM((H * _BPW,), jnp.int32),
            pltpu.VMEM((H * _BPW,), jnp.int32),
            pltpu.VMEM((_BPW,), jnp.float32),
            pltpu.VMEM((_BPW,), jnp.float32),
            pltpu.VMEM((H * _BPW,), jnp.float32),
            pltpu.VMEM((H * _BPW,), jnp.float32),
            pltpu.VMEM((H * _BPW,), jnp.float32),
            pltpu.VMEM((_BPW,), jnp.float32),
            pltpu.SemaphoreType.DMA,
        ],
    )
    return run(user_id, item_id, pvf, byf, fvf, us, is_, pvs, bys, fvs)


def kernel(user_id, item_id, pv_history, buy_history, fav_history,
           user_table, item_table, fc_w, fc_b):
    w = fc_w[0]
    zrow = jnp.zeros((1, D), jnp.float32)
    wu = jnp.concatenate([w[0:128][None, :]] + [zrow] * 7, axis=0)
    wi = jnp.concatenate(
        [zrow, w[128:256][None, :], w[256:384][None, :],
         w[384:512][None, :], w[512:640][None, :], zrow, zrow, zrow], axis=0)
    bias_row = jnp.zeros((1, 8), jnp.float32).at[0, 0].set(fc_b[0])
    us, is_, pvs, bys, fvs = _scores_tc(user_table, item_table, wu, wi, bias_row)
    return _sc_pool(user_id, item_id,
                    pv_history.reshape(-1), buy_history.reshape(-1),
                    fav_history.reshape(-1), us, is_, pvs, bys, fvs)


# speedup vs baseline: 1.2275x; 1.1796x over previous
import jax
import jax.numpy as jnp
from jax import lax
from jax.experimental import pallas as pl
from jax.experimental.pallas import tpu as pltpu
from jax.experimental.pallas import tpu_sc as plsc

D = 128
H = 50
B = 4096
N_ROWS = 100000

_SC_PARAMS = pltpu.CompilerParams(needs_layout_passes=False)


_R_BLK = 10240


def _scores_body(wi_ref, i_ref, o0_ref, o1_ref, o2_ref, o3_ref):
    it = i_ref[...]
    dn = (((1,), (1,)), ((), ()))
    res = lax.dot_general(wi_ref[...], it, dn,
                          preferred_element_type=jnp.float32)
    o0_ref[...] = res[0, :]
    o1_ref[...] = res[1, :]
    o2_ref[...] = res[2, :]
    o3_ref[...] = res[3, :]


def _scores_tc(item_table, wi):
    n_blk = (N_ROWS + _R_BLK - 1) // _R_BLK
    one_d = pl.BlockSpec((_R_BLK,), lambda i: (i,))
    return pl.pallas_call(
        _scores_body,
        grid=(n_blk,),
        in_specs=[
            pl.BlockSpec((8, D), lambda i: (0, 0)),
            pl.BlockSpec((_R_BLK, D), lambda i: (i, 0)),
        ],
        out_specs=[one_d] * 4,
        out_shape=[jax.ShapeDtypeStruct((N_ROWS,), jnp.float32)] * 4,
    )(wi, item_table)


_NC = 2
_NS = 16
_NW = _NC * _NS
_BPW = B // _NW
_NG = _BPW // 16


def _user_body(uid_hbm, ut_hbm, wb_hbm, out_hbm,
               uidx, wscr, rows, parts, obuf, sem):
    wid = lax.axis_index("s") * _NC + lax.axis_index("c")
    base = wid * _BPW
    pltpu.sync_copy(uid_hbm.at[pl.ds(base, _BPW)], uidx)
    pltpu.sync_copy(wb_hbm, wscr)
    pltpu.async_copy(ut_hbm.at[uidx], rows, sem).wait()
    wv = [wscr[pl.ds(k * 16, 16)] for k in range(8)]

    def rbody(r, carry):
        acc = rows[r, pl.ds(0, 16)] * wv[0]
        for k in range(1, 8):
            acc = acc + rows[r, pl.ds(k * 16, 16)] * wv[k]
        parts[pl.ds(r * 16, 16)] = acc
        return carry

    lax.fori_loop(0, _BPW, rbody, 0)
    bias_vec = wscr[pl.ds(128, 16)]
    lanes = lax.iota(jnp.int32, 16)
    for g in range(_NG):
        bvec = (g * 16 + lanes) * 16
        acc = plsc.load_gather(parts, [bvec])
        for k in range(1, 16):
            acc = acc + plsc.load_gather(parts, [bvec + k])
        obuf[pl.ds(g * 16, 16)] = acc + bias_vec
    pltpu.sync_copy(obuf, out_hbm.at[pl.ds(base, _BPW)])


def _user_sc(user_id, user_table, w0b):
    mesh = plsc.VectorSubcoreMesh(core_axis_name="c", subcore_axis_name="s",
                                  num_cores=_NC, num_subcores=_NS)
    run = pl.kernel(
        _user_body,
        jax.ShapeDtypeStruct((B,), jnp.float32),
        mesh=mesh,
        compiler_params=_SC_PARAMS,
        scratch_types=[
            pltpu.VMEM((_BPW,), jnp.int32),
            pltpu.VMEM((160,), jnp.float32),
            pltpu.VMEM((_BPW, D), jnp.float32),
            pltpu.VMEM((_BPW * 16,), jnp.float32),
            pltpu.VMEM((_BPW,), jnp.float32),
            pltpu.SemaphoreType.DMA,
        ],
    )
    return run(user_id, user_table, w0b)


def _sc_body(iid_hbm, pvf_hbm, byf_hbm, fvf_hbm,
             ub_hbm, is_hbm, pvs_hbm, bys_hbm, fvs_hbm,
             out_hbm,
             iidx, pvidx, byidx, fvidx,
             uval, ival, pvval, byval, fvval, obuf, sem):
    wid = lax.axis_index("s") * _NC + lax.axis_index("c")
    base = wid * _BPW
    hbase = base * H
    pltpu.sync_copy(iid_hbm.at[pl.ds(base, _BPW)], iidx)
    pltpu.sync_copy(pvf_hbm.at[pl.ds(hbase, _BPW * H)], pvidx)
    pltpu.sync_copy(byf_hbm.at[pl.ds(hbase, _BPW * H)], byidx)
    pltpu.sync_copy(fvf_hbm.at[pl.ds(hbase, _BPW * H)], fvidx)
    pltpu.sync_copy(ub_hbm.at[pl.ds(base, _BPW)], uval)
    c1 = pltpu.async_copy(is_hbm.at[iidx], ival, sem)
    c2 = pltpu.async_copy(pvs_hbm.at[pvidx], pvval, sem)
    c3 = pltpu.async_copy(bys_hbm.at[byidx], byval, sem)
    c4 = pltpu.async_copy(fvs_hbm.at[fvidx], fvval, sem)
    c1.wait(); c2.wait(); c3.wait(); c4.wait()
    inv_h = jnp.float32(1.0 / H)
    lanes = lax.iota(jnp.int32, 16)
    for g in range(_NG):
        sl = pl.ds(g * 16, 16)
        bvec = (g * 16 + lanes) * H

        def hbody(h, acc):
            idx = bvec + h
            return (acc
                    + plsc.load_gather(pvval, [idx])
                    + plsc.load_gather(byval, [idx])
                    + plsc.load_gather(fvval, [idx]))

        acc = lax.fori_loop(0, H, hbody, jnp.zeros((16,), jnp.float32))
        x = uval[sl] + ival[sl] + acc * inv_h
        obuf[sl] = 1.0 / (1.0 + jnp.exp(-x))
    pltpu.sync_copy(obuf, out_hbm.at[pl.ds(base, _BPW)])


def _sc_pool(item_id, pvf, byf, fvf, ub, is_, pvs, bys, fvs):
    mesh = plsc.VectorSubcoreMesh(core_axis_name="c", subcore_axis_name="s",
                                  num_cores=_NC, num_subcores=_NS)
    run = pl.kernel(
        _sc_body,
        jax.ShapeDtypeStruct((B,), jnp.float32),
        mesh=mesh,
        compiler_params=_SC_PARAMS,
        scratch_types=[
            pltpu.VMEM((_BPW,), jnp.int32),
            pltpu.VMEM((H * _BPW,), jnp.int32),
            pltpu.VMEM((H * _BPW,), jnp.int32),
            pltpu.VMEM((H * _BPW,), jnp.int32),
            pltpu.VMEM((_BPW,), jnp.float32),
            pltpu.VMEM((_BPW,), jnp.float32),
            pltpu.VMEM((H * _BPW,), jnp.float32),
            pltpu.VMEM((H * _BPW,), jnp.float32),
            pltpu.VMEM((H * _BPW,), jnp.float32),
            pltpu.VMEM((_BPW,), jnp.float32),
            pltpu.SemaphoreType.DMA,
        ],
    )
    return run(item_id, pvf, byf, fvf, ub, is_, pvs, bys, fvs)


def kernel(user_id, item_id, pv_history, buy_history, fav_history,
           user_table, item_table, fc_w, fc_b):
    w = fc_w[0]
    zrow = jnp.zeros((1, D), jnp.float32)
    wi = jnp.concatenate(
        [w[128:256][None, :], w[256:384][None, :], w[384:512][None, :],
         w[512:640][None, :], zrow, zrow, zrow, zrow], axis=0)
    w0b = jnp.concatenate([w[0:128], jnp.full((16,), fc_b[0], jnp.float32),
                           jnp.zeros((16,), jnp.float32)])
    ub = _user_sc(user_id, user_table, w0b)
    is_, pvs, bys, fvs = _scores_tc(item_table, wi)
    return _sc_pool(item_id,
                    pv_history.reshape(-1), buy_history.reshape(-1),
                    fav_history.reshape(-1), ub, is_, pvs, bys, fvs)


# speedup vs baseline: 1.2394x; 1.0097x over previous
import jax
import jax.numpy as jnp
from jax import lax
from jax.experimental import pallas as pl
from jax.experimental.pallas import tpu as pltpu
from jax.experimental.pallas import tpu_sc as plsc

D = 128
H = 50
B = 4096
N_ROWS = 100000

_SC_PARAMS = pltpu.CompilerParams(needs_layout_passes=False)


_R_BLK = 20480


def _scores_body(wi_ref, i_ref, o0_ref, o1_ref, o2_ref, o3_ref):
    it = i_ref[...]
    dn = (((1,), (1,)), ((), ()))
    res = lax.dot_general(wi_ref[...], it, dn,
                          preferred_element_type=jnp.float32)
    o0_ref[...] = res[0, :]
    o1_ref[...] = res[1, :]
    o2_ref[...] = res[2, :]
    o3_ref[...] = res[3, :]


def _scores_tc(item_table, wi):
    n_blk = (N_ROWS + _R_BLK - 1) // _R_BLK
    one_d = pl.BlockSpec((_R_BLK,), lambda i: (i,))
    return pl.pallas_call(
        _scores_body,
        grid=(n_blk,),
        in_specs=[
            pl.BlockSpec((8, D), lambda i: (0, 0)),
            pl.BlockSpec((_R_BLK, D), lambda i: (i, 0)),
        ],
        out_specs=[one_d] * 4,
        out_shape=[jax.ShapeDtypeStruct((N_ROWS,), jnp.float32)] * 4,
    )(wi, item_table)


_NC = 2
_NS = 16
_NW = _NC * _NS
_BPW = B // _NW
_NG = _BPW // 16


def _user_body(uid_hbm, ut_hbm, wb_hbm, out_hbm,
               uidx, wscr, rows, parts, obuf, sem):
    wid = lax.axis_index("s") * _NC + lax.axis_index("c")
    base = wid * _BPW
    pltpu.sync_copy(uid_hbm.at[pl.ds(base, _BPW)], uidx)
    pltpu.sync_copy(wb_hbm, wscr)
    pltpu.async_copy(ut_hbm.at[uidx], rows, sem).wait()
    wv = [wscr[pl.ds(k * 16, 16)] for k in range(8)]

    def rbody(r, carry):
        acc = rows[r, pl.ds(0, 16)] * wv[0]
        for k in range(1, 8):
            acc = acc + rows[r, pl.ds(k * 16, 16)] * wv[k]
        parts[pl.ds(r * 16, 16)] = acc
        return carry

    lax.fori_loop(0, _BPW, rbody, 0)
    bias_vec = wscr[pl.ds(128, 16)]
    lanes = lax.iota(jnp.int32, 16)
    for g in range(_NG):
        bvec = (g * 16 + lanes) * 16
        acc = plsc.load_gather(parts, [bvec])
        for k in range(1, 16):
            acc = acc + plsc.load_gather(parts, [bvec + k])
        obuf[pl.ds(g * 16, 16)] = acc + bias_vec
    pltpu.sync_copy(obuf, out_hbm.at[pl.ds(base, _BPW)])


def _user_sc(user_id, user_table, w0b):
    mesh = plsc.VectorSubcoreMesh(core_axis_name="c", subcore_axis_name="s",
                                  num_cores=_NC, num_subcores=_NS)
    run = pl.kernel(
        _user_body,
        jax.ShapeDtypeStruct((B,), jnp.float32),
        mesh=mesh,
        compiler_params=_SC_PARAMS,
        scratch_types=[
            pltpu.VMEM((_BPW,), jnp.int32),
            pltpu.VMEM((160,), jnp.float32),
            pltpu.VMEM((_BPW, D), jnp.float32),
            pltpu.VMEM((_BPW * 16,), jnp.float32),
            pltpu.VMEM((_BPW,), jnp.float32),
            pltpu.SemaphoreType.DMA,
        ],
    )
    return run(user_id, user_table, w0b)


def _sc_body(iid_hbm, pvf_hbm, byf_hbm, fvf_hbm,
             ub_hbm, is_hbm, pvs_hbm, bys_hbm, fvs_hbm,
             out_hbm,
             iidx, pvidx, byidx, fvidx,
             uval, ival, pvval, byval, fvval, obuf, sem):
    wid = lax.axis_index("s") * _NC + lax.axis_index("c")
    base = wid * _BPW
    hbase = base * H
    pltpu.sync_copy(iid_hbm.at[pl.ds(base, _BPW)], iidx)
    pltpu.sync_copy(pvf_hbm.at[pl.ds(hbase, _BPW * H)], pvidx)
    pltpu.sync_copy(byf_hbm.at[pl.ds(hbase, _BPW * H)], byidx)
    pltpu.sync_copy(fvf_hbm.at[pl.ds(hbase, _BPW * H)], fvidx)
    pltpu.sync_copy(ub_hbm.at[pl.ds(base, _BPW)], uval)
    c1 = pltpu.async_copy(is_hbm.at[iidx], ival, sem)
    c2 = pltpu.async_copy(pvs_hbm.at[pvidx], pvval, sem)
    c3 = pltpu.async_copy(bys_hbm.at[byidx], byval, sem)
    c4 = pltpu.async_copy(fvs_hbm.at[fvidx], fvval, sem)
    c1.wait(); c2.wait(); c3.wait(); c4.wait()
    inv_h = jnp.float32(1.0 / H)
    lanes = lax.iota(jnp.int32, 16)
    for g in range(_NG):
        sl = pl.ds(g * 16, 16)
        bvec = (g * 16 + lanes) * H

        def hbody(h, acc):
            idx = bvec + h
            return (acc
                    + plsc.load_gather(pvval, [idx])
                    + plsc.load_gather(byval, [idx])
                    + plsc.load_gather(fvval, [idx]))

        acc = lax.fori_loop(0, H, hbody, jnp.zeros((16,), jnp.float32))
        x = uval[sl] + ival[sl] + acc * inv_h
        obuf[sl] = 1.0 / (1.0 + jnp.exp(-x))
    pltpu.sync_copy(obuf, out_hbm.at[pl.ds(base, _BPW)])


def _sc_pool(item_id, pvf, byf, fvf, ub, is_, pvs, bys, fvs):
    mesh = plsc.VectorSubcoreMesh(core_axis_name="c", subcore_axis_name="s",
                                  num_cores=_NC, num_subcores=_NS)
    run = pl.kernel(
        _sc_body,
        jax.ShapeDtypeStruct((B,), jnp.float32),
        mesh=mesh,
        compiler_params=_SC_PARAMS,
        scratch_types=[
            pltpu.VMEM((_BPW,), jnp.int32),
            pltpu.VMEM((H * _BPW,), jnp.int32),
            pltpu.VMEM((H * _BPW,), jnp.int32),
            pltpu.VMEM((H * _BPW,), jnp.int32),
            pltpu.VMEM((_BPW,), jnp.float32),
            pltpu.VMEM((_BPW,), jnp.float32),
            pltpu.VMEM((H * _BPW,), jnp.float32),
            pltpu.VMEM((H * _BPW,), jnp.float32),
            pltpu.VMEM((H * _BPW,), jnp.float32),
            pltpu.VMEM((_BPW,), jnp.float32),
            pltpu.SemaphoreType.DMA,
        ],
    )
    return run(item_id, pvf, byf, fvf, ub, is_, pvs, bys, fvs)


def kernel(user_id, item_id, pv_history, buy_history, fav_history,
           user_table, item_table, fc_w, fc_b):
    w = fc_w[0]
    zrow = jnp.zeros((1, D), jnp.float32)
    wi = jnp.concatenate(
        [w[128:256][None, :], w[256:384][None, :], w[384:512][None, :],
         w[512:640][None, :], zrow, zrow, zrow, zrow], axis=0)
    w0b = jnp.concatenate([w[0:128], jnp.full((16,), fc_b[0], jnp.float32),
                           jnp.zeros((16,), jnp.float32)])
    ub = _user_sc(user_id, user_table, w0b)
    is_, pvs, bys, fvs = _scores_tc(item_table, wi)
    return _sc_pool(item_id,
                    pv_history.reshape(-1), buy_history.reshape(-1),
                    fav_history.reshape(-1), ub, is_, pvs, bys, fvs)


# speedup vs baseline: 1.2416x; 1.0018x over previous
import jax
import jax.numpy as jnp
from jax import lax
from jax.experimental import pallas as pl
from jax.experimental.pallas import tpu as pltpu
from jax.experimental.pallas import tpu_sc as plsc

D = 128
H = 50
B = 4096
N_ROWS = 100000

_SC_PARAMS = pltpu.CompilerParams(needs_layout_passes=False)


_R_BLK = 10240


def _scores_body(wi_ref, ia_ref, ib_ref, o0_ref, o1_ref, o2_ref, o3_ref):
    dn = (((1,), (1,)), ((), ()))
    wv = wi_ref[...]
    ra = lax.dot_general(wv, ia_ref[...], dn,
                         preferred_element_type=jnp.float32)
    rb = lax.dot_general(wv, ib_ref[...], dn,
                         preferred_element_type=jnp.float32)
    lo = pl.ds(0, _R_BLK)
    hi = pl.ds(_R_BLK, _R_BLK)
    o0_ref[lo] = ra[0, :]; o0_ref[hi] = rb[0, :]
    o1_ref[lo] = ra[1, :]; o1_ref[hi] = rb[1, :]
    o2_ref[lo] = ra[2, :]; o2_ref[hi] = rb[2, :]
    o3_ref[lo] = ra[3, :]; o3_ref[hi] = rb[3, :]


def _scores_tc(item_table, wi):
    n_blk = (N_ROWS + 2 * _R_BLK - 1) // (2 * _R_BLK)
    one_d = pl.BlockSpec((2 * _R_BLK,), lambda i: (i,))
    return pl.pallas_call(
        _scores_body,
        grid=(n_blk,),
        in_specs=[
            pl.BlockSpec((8, D), lambda i: (0, 0)),
            pl.BlockSpec((_R_BLK, D), lambda i: (2 * i, 0)),
            pl.BlockSpec((_R_BLK, D), lambda i: (2 * i + 1, 0)),
        ],
        out_specs=[one_d] * 4,
        out_shape=[jax.ShapeDtypeStruct((N_ROWS,), jnp.float32)] * 4,
    )(wi, item_table, item_table)


_NC = 2
_NS = 16
_NW = _NC * _NS
_BPW = B // _NW
_NG = _BPW // 16


def _user_body(uid_hbm, ut_hbm, wb_hbm, out_hbm,
               uidx, wscr, rows, parts, obuf, sem):
    wid = lax.axis_index("s") * _NC + lax.axis_index("c")
    base = wid * _BPW
    pltpu.sync_copy(uid_hbm.at[pl.ds(base, _BPW)], uidx)
    pltpu.sync_copy(wb_hbm, wscr)
    pltpu.async_copy(ut_hbm.at[uidx], rows, sem).wait()
    wv = [wscr[pl.ds(k * 16, 16)] for k in range(8)]

    def rbody(r, carry):
        acc = rows[r, pl.ds(0, 16)] * wv[0]
        for k in range(1, 8):
            acc = acc + rows[r, pl.ds(k * 16, 16)] * wv[k]
        parts[pl.ds(r * 16, 16)] = acc
        return carry

    lax.fori_loop(0, _BPW, rbody, 0)
    bias_vec = wscr[pl.ds(128, 16)]
    lanes = lax.iota(jnp.int32, 16)
    for g in range(_NG):
        bvec = (g * 16 + lanes) * 16
        acc = plsc.load_gather(parts, [bvec])
        for k in range(1, 16):
            acc = acc + plsc.load_gather(parts, [bvec + k])
        obuf[pl.ds(g * 16, 16)] = acc + bias_vec
    pltpu.sync_copy(obuf, out_hbm.at[pl.ds(base, _BPW)])


def _user_sc(user_id, user_table, w0b):
    mesh = plsc.VectorSubcoreMesh(core_axis_name="c", subcore_axis_name="s",
                                  num_cores=_NC, num_subcores=_NS)
    run = pl.kernel(
        _user_body,
        jax.ShapeDtypeStruct((B,), jnp.float32),
        mesh=mesh,
        compiler_params=_SC_PARAMS,
        scratch_types=[
            pltpu.VMEM((_BPW,), jnp.int32),
            pltpu.VMEM((160,), jnp.float32),
            pltpu.VMEM((_BPW, D), jnp.float32),
            pltpu.VMEM((_BPW * 16,), jnp.float32),
            pltpu.VMEM((_BPW,), jnp.float32),
            pltpu.SemaphoreType.DMA,
        ],
    )
    return run(user_id, user_table, w0b)


def _sc_body(iid_hbm, pvf_hbm, byf_hbm, fvf_hbm,
             ub_hbm, is_hbm, pvs_hbm, bys_hbm, fvs_hbm,
             out_hbm,
             iidx, pvidx, byidx, fvidx,
             uval, ival, pvval, byval, fvval, obuf, sem):
    wid = lax.axis_index("s") * _NC + lax.axis_index("c")
    base = wid * _BPW
    hbase = base * H
    pltpu.sync_copy(iid_hbm.at[pl.ds(base, _BPW)], iidx)
    pltpu.sync_copy(pvf_hbm.at[pl.ds(hbase, _BPW * H)], pvidx)
    pltpu.sync_copy(byf_hbm.at[pl.ds(hbase, _BPW * H)], byidx)
    pltpu.sync_copy(fvf_hbm.at[pl.ds(hbase, _BPW * H)], fvidx)
    pltpu.sync_copy(ub_hbm.at[pl.ds(base, _BPW)], uval)
    c1 = pltpu.async_copy(is_hbm.at[iidx], ival, sem)
    c2 = pltpu.async_copy(pvs_hbm.at[pvidx], pvval, sem)
    c3 = pltpu.async_copy(bys_hbm.at[byidx], byval, sem)
    c4 = pltpu.async_copy(fvs_hbm.at[fvidx], fvval, sem)
    c1.wait(); c2.wait(); c3.wait(); c4.wait()
    inv_h = jnp.float32(1.0 / H)
    lanes = lax.iota(jnp.int32, 16)
    for g in range(_NG):
        sl = pl.ds(g * 16, 16)
        bvec = (g * 16 + lanes) * H

        def hbody(h, acc):
            idx = bvec + h
            return (acc
                    + plsc.load_gather(pvval, [idx])
                    + plsc.load_gather(byval, [idx])
                    + plsc.load_gather(fvval, [idx]))

        acc = lax.fori_loop(0, H, hbody, jnp.zeros((16,), jnp.float32))
        x = uval[sl] + ival[sl] + acc * inv_h
        obuf[sl] = 1.0 / (1.0 + jnp.exp(-x))
    pltpu.sync_copy(obuf, out_hbm.at[pl.ds(base, _BPW)])


def _sc_pool(item_id, pvf, byf, fvf, ub, is_, pvs, bys, fvs):
    mesh = plsc.VectorSubcoreMesh(core_axis_name="c", subcore_axis_name="s",
                                  num_cores=_NC, num_subcores=_NS)
    run = pl.kernel(
        _sc_body,
        jax.ShapeDtypeStruct((B,), jnp.float32),
        mesh=mesh,
        compiler_params=_SC_PARAMS,
        scratch_types=[
            pltpu.VMEM((_BPW,), jnp.int32),
            pltpu.VMEM((H * _BPW,), jnp.int32),
            pltpu.VMEM((H * _BPW,), jnp.int32),
            pltpu.VMEM((H * _BPW,), jnp.int32),
            pltpu.VMEM((_BPW,), jnp.float32),
            pltpu.VMEM((_BPW,), jnp.float32),
            pltpu.VMEM((H * _BPW,), jnp.float32),
            pltpu.VMEM((H * _BPW,), jnp.float32),
            pltpu.VMEM((H * _BPW,), jnp.float32),
            pltpu.VMEM((_BPW,), jnp.float32),
            pltpu.SemaphoreType.DMA,
        ],
    )
    return run(item_id, pvf, byf, fvf, ub, is_, pvs, bys, fvs)


def kernel(user_id, item_id, pv_history, buy_history, fav_history,
           user_table, item_table, fc_w, fc_b):
    w = fc_w[0]
    zrow = jnp.zeros((1, D), jnp.float32)
    wi = jnp.concatenate(
        [w[128:256][None, :], w[256:384][None, :], w[384:512][None, :],
         w[512:640][None, :], zrow, zrow, zrow, zrow], axis=0)
    w0b = jnp.concatenate([w[0:128], jnp.full((16,), fc_b[0], jnp.float32),
                           jnp.zeros((16,), jnp.float32)])
    ub = _user_sc(user_id, user_table, w0b)
    is_, pvs, bys, fvs = _scores_tc(item_table, wi)
    return _sc_pool(item_id,
                    pv_history.reshape(-1), buy_history.reshape(-1),
                    fav_history.reshape(-1), ub, is_, pvs, bys, fvs)


# speedup vs baseline: 1.2582x; 1.0134x over previous
import jax
import jax.numpy as jnp
from jax import lax
from jax.experimental import pallas as pl
from jax.experimental.pallas import tpu as pltpu
from jax.experimental.pallas import tpu_sc as plsc

D = 128
H = 50
B = 4096
N_ROWS = 100000

_SC_PARAMS = pltpu.CompilerParams(needs_layout_passes=False)


_R_BLK = 10240


def _scores_body(wi_ref, ia_ref, ib_ref, pv2_ref, by2_ref, fv2_ref,
                 o0_ref, o1_ref, o2_ref, o3_ref,
                 pvo_ref, byo_ref, fvo_ref):
    dn = (((1,), (1,)), ((), ()))
    wv = wi_ref[...]
    ra = lax.dot_general(wv, ia_ref[...], dn,
                         preferred_element_type=jnp.float32)
    rb = lax.dot_general(wv, ib_ref[...], dn,
                         preferred_element_type=jnp.float32)
    lo = pl.ds(0, _R_BLK)
    hi = pl.ds(_R_BLK, _R_BLK)
    o0_ref[lo] = ra[0, :]; o0_ref[hi] = rb[0, :]
    o1_ref[lo] = ra[1, :]; o1_ref[hi] = rb[1, :]
    o2_ref[lo] = ra[2, :]; o2_ref[hi] = rb[2, :]
    o3_ref[lo] = ra[3, :]; o3_ref[hi] = rb[3, :]

    @pl.when(pl.program_id(0) == 0)
    def _():
        pvo_ref[...] = pv2_ref[...].T
        byo_ref[...] = by2_ref[...].T
        fvo_ref[...] = fv2_ref[...].T


def _scores_tc(item_table, wi, pv2, by2, fv2):
    n_blk = (N_ROWS + 2 * _R_BLK - 1) // (2 * _R_BLK)
    one_d = pl.BlockSpec((2 * _R_BLK,), lambda i: (i,))
    full2 = pl.BlockSpec((B, H), lambda i: (0, 0))
    full2t = pl.BlockSpec((H, B), lambda i: (0, 0))
    return pl.pallas_call(
        _scores_body,
        grid=(n_blk,),
        in_specs=[
            pl.BlockSpec((8, D), lambda i: (0, 0)),
            pl.BlockSpec((_R_BLK, D), lambda i: (2 * i, 0)),
            pl.BlockSpec((_R_BLK, D), lambda i: (2 * i + 1, 0)),
            full2, full2, full2,
        ],
        out_specs=[one_d] * 4 + [full2t] * 3,
        out_shape=[jax.ShapeDtypeStruct((N_ROWS,), jnp.float32)] * 4
        + [jax.ShapeDtypeStruct((H, B), jnp.int32)] * 3,
    )(wi, item_table, item_table, pv2, by2, fv2)


_NC = 2
_NS = 16
_NW = _NC * _NS
_BPW = B // _NW
_NG = _BPW // 16


def _user_body(uid_hbm, ut_hbm, wb_hbm, out_hbm,
               uidx, wscr, rows, parts, obuf, sem):
    wid = lax.axis_index("s") * _NC + lax.axis_index("c")
    base = wid * _BPW
    pltpu.sync_copy(uid_hbm.at[pl.ds(base, _BPW)], uidx)
    pltpu.sync_copy(wb_hbm, wscr)
    pltpu.async_copy(ut_hbm.at[uidx], rows, sem).wait()
    wv = [wscr[pl.ds(k * 16, 16)] for k in range(8)]

    def rbody(r, carry):
        acc = rows[r, pl.ds(0, 16)] * wv[0]
        for k in range(1, 8):
            acc = acc + rows[r, pl.ds(k * 16, 16)] * wv[k]
        parts[pl.ds(r * 16, 16)] = acc
        return carry

    lax.fori_loop(0, _BPW, rbody, 0)
    bias_vec = wscr[pl.ds(128, 16)]
    lanes = lax.iota(jnp.int32, 16)
    for g in range(_NG):
        bvec = (g * 16 + lanes) * 16
        acc = plsc.load_gather(parts, [bvec])
        for k in range(1, 16):
            acc = acc + plsc.load_gather(parts, [bvec + k])
        obuf[pl.ds(g * 16, 16)] = acc + bias_vec
    pltpu.sync_copy(obuf, out_hbm.at[pl.ds(base, _BPW)])


def _user_sc(user_id, user_table, w0b):
    mesh = plsc.VectorSubcoreMesh(core_axis_name="c", subcore_axis_name="s",
                                  num_cores=_NC, num_subcores=_NS)
    run = pl.kernel(
        _user_body,
        jax.ShapeDtypeStruct((B,), jnp.float32),
        mesh=mesh,
        compiler_params=_SC_PARAMS,
        scratch_types=[
            pltpu.VMEM((_BPW,), jnp.int32),
            pltpu.VMEM((160,), jnp.float32),
            pltpu.VMEM((_BPW, D), jnp.float32),
            pltpu.VMEM((_BPW * 16,), jnp.float32),
            pltpu.VMEM((_BPW,), jnp.float32),
            pltpu.SemaphoreType.DMA,
        ],
    )
    return run(user_id, user_table, w0b)


def _sc_body(iid_hbm, pvf_hbm, byf_hbm, fvf_hbm,
             ub_hbm, is_hbm, pvs_hbm, bys_hbm, fvs_hbm,
             out_hbm,
             iidx, pvidx, byidx, fvidx,
             uval, ival, pvval, byval, fvval, obuf, sem):
    wid = lax.axis_index("s") * _NC + lax.axis_index("c")
    base = wid * _BPW
    pltpu.sync_copy(iid_hbm.at[pl.ds(base, _BPW)], iidx)
    stage = []
    for h in range(H):
        dst = pl.ds(h * _BPW, _BPW)
        stage.append(pltpu.async_copy(
            pvf_hbm.at[h, pl.ds(base, _BPW)], pvidx.at[dst], sem))
        stage.append(pltpu.async_copy(
            byf_hbm.at[h, pl.ds(base, _BPW)], byidx.at[dst], sem))
        stage.append(pltpu.async_copy(
            fvf_hbm.at[h, pl.ds(base, _BPW)], fvidx.at[dst], sem))
    pltpu.sync_copy(ub_hbm.at[pl.ds(base, _BPW)], uval)
    for c in stage:
        c.wait()
    c1 = pltpu.async_copy(is_hbm.at[iidx], ival, sem)
    c2 = pltpu.async_copy(pvs_hbm.at[pvidx], pvval, sem)
    c3 = pltpu.async_copy(bys_hbm.at[byidx], byval, sem)
    c4 = pltpu.async_copy(fvs_hbm.at[fvidx], fvval, sem)
    c1.wait(); c2.wait(); c3.wait(); c4.wait()
    inv_h = jnp.float32(1.0 / H)
    for g in range(_NG):
        sl = pl.ds(g * 16, 16)

        def hbody(h, acc):
            hsl = pl.ds(h * _BPW + g * 16, 16)
            return acc + pvval[hsl] + byval[hsl] + fvval[hsl]

        acc = lax.fori_loop(0, H, hbody, jnp.zeros((16,), jnp.float32))
        x = uval[sl] + ival[sl] + acc * inv_h
        obuf[sl] = 1.0 / (1.0 + jnp.exp(-x))
    pltpu.sync_copy(obuf, out_hbm.at[pl.ds(base, _BPW)])


def _sc_pool(item_id, pvf, byf, fvf, ub, is_, pvs, bys, fvs):
    mesh = plsc.VectorSubcoreMesh(core_axis_name="c", subcore_axis_name="s",
                                  num_cores=_NC, num_subcores=_NS)
    run = pl.kernel(
        _sc_body,
        jax.ShapeDtypeStruct((B,), jnp.float32),
        mesh=mesh,
        compiler_params=_SC_PARAMS,
        scratch_types=[
            pltpu.VMEM((_BPW,), jnp.int32),
            pltpu.VMEM((H * _BPW,), jnp.int32),
            pltpu.VMEM((H * _BPW,), jnp.int32),
            pltpu.VMEM((H * _BPW,), jnp.int32),
            pltpu.VMEM((_BPW,), jnp.float32),
            pltpu.VMEM((_BPW,), jnp.float32),
            pltpu.VMEM((H * _BPW,), jnp.float32),
            pltpu.VMEM((H * _BPW,), jnp.float32),
            pltpu.VMEM((H * _BPW,), jnp.float32),
            pltpu.VMEM((_BPW,), jnp.float32),
            pltpu.SemaphoreType.DMA,
        ],
    )
    return run(item_id, pvf, byf, fvf, ub, is_, pvs, bys, fvs)


def kernel(user_id, item_id, pv_history, buy_history, fav_history,
           user_table, item_table, fc_w, fc_b):
    w = fc_w[0]
    zrow = jnp.zeros((1, D), jnp.float32)
    wi = jnp.concatenate(
        [w[128:256][None, :], w[256:384][None, :], w[384:512][None, :],
         w[512:640][None, :], zrow, zrow, zrow, zrow], axis=0)
    w0b = jnp.concatenate([w[0:128], jnp.full((16,), fc_b[0], jnp.float32),
                           jnp.zeros((16,), jnp.float32)])
    ub = _user_sc(user_id, user_table, w0b)
    is_, pvs, bys, fvs, pvf, byf, fvf = _scores_tc(
        item_table, wi, pv_history, buy_history, fav_history)
    return _sc_pool(item_id, pvf, byf, fvf, ub, is_, pvs, bys, fvs)
